# baseline (speedup 1.0000x reference)
import jax
import jax.numpy as jnp
from jax.experimental import pallas as pl
from jax.experimental.pallas import tpu as pltpu

B, S, H, D = 8, 2048, 8, 128
S_CHUNKS = 4
CS = S // S_CHUNKS


def _kv_append_kernel(len_ref, ck, cv, kt, vt, ok, ov, sem_big, sem_tok):
    for b in range(B):
        for c in range(S_CHUNKS):
            sl = pl.ds(c * CS, CS)
            pltpu.make_async_copy(ck.at[b, sl], ok.at[b, sl], sem_big).start()
            pltpu.make_async_copy(cv.at[b, sl], ov.at[b, sl], sem_big).start()
    for b in range(B):
        for c in range(S_CHUNKS):
            sl = pl.ds(c * CS, CS)
            pltpu.make_async_copy(ck.at[b, sl], ok.at[b, sl], sem_big).wait()
            pltpu.make_async_copy(cv.at[b, sl], ov.at[b, sl], sem_big).wait()
    for b in range(B):
        l = len_ref[b]
        pltpu.make_async_copy(kt.at[b, 0], ok.at[b, l], sem_tok).start()
        pltpu.make_async_copy(vt.at[b, 0], ov.at[b, l], sem_tok).start()
    for b in range(B):
        l = len_ref[b]
        pltpu.make_async_copy(kt.at[b, 0], ok.at[b, l], sem_tok).wait()
        pltpu.make_async_copy(vt.at[b, 0], ov.at[b, l], sem_tok).wait()


def kernel(cached_key, cached_value, key_token, value_token, lengths):
    out_sds = jax.ShapeDtypeStruct((B, S, H, D), jnp.float32)
    new_key, new_value = pl.pallas_call(
        _kv_append_kernel,
        grid=(),
        in_specs=[
            pl.BlockSpec(memory_space=pltpu.SMEM),
            pl.BlockSpec(memory_space=pltpu.MemorySpace.HBM),
            pl.BlockSpec(memory_space=pltpu.MemorySpace.HBM),
            pl.BlockSpec(memory_space=pltpu.MemorySpace.HBM),
            pl.BlockSpec(memory_space=pltpu.MemorySpace.HBM),
        ],
        out_specs=[
            pl.BlockSpec(memory_space=pltpu.MemorySpace.HBM),
            pl.BlockSpec(memory_space=pltpu.MemorySpace.HBM),
        ],
        out_shape=[out_sds, out_sds],
        scratch_shapes=[pltpu.SemaphoreType.DMA, pltpu.SemaphoreType.DMA],
    )(lengths, cached_key, cached_value, key_token, value_token)
    return (new_key, new_value)


# speedup vs baseline: 44.3061x; 44.3061x over previous
import jax
import jax.numpy as jnp
from jax.experimental import pallas as pl
from jax.experimental.pallas import tpu as pltpu

B, S, H, D = 8, 2048, 8, 128
S_CHUNKS = 8
CS = S // S_CHUNKS


def _kv_append_kernel(len_ref, ck, cv, kt, vt, ok, ov):
    b = pl.program_id(0)
    c = pl.program_id(1)
    ok[...] = ck[...]
    ov[...] = cv[...]
    l = len_ref[b]
    base = c * CS

    @pl.when((l >= base) & (l < base + CS))
    def _():
        r = l - base
        ok[0, pl.ds(r, 1)] = kt[pl.ds(b, 1), 0]
        ov[0, pl.ds(r, 1)] = vt[pl.ds(b, 1), 0]


def kernel(cached_key, cached_value, key_token, value_token, lengths):
    out_sds = jax.ShapeDtypeStruct((B, S, H, D), jnp.float32)
    cache_spec = pl.BlockSpec((1, CS, H, D), lambda b, c: (b, c, 0, 0))
    token_spec = pl.BlockSpec((B, 1, H, D), lambda b, c: (0, 0, 0, 0))
    new_key, new_value = pl.pallas_call(
        _kv_append_kernel,
        grid=(B, S_CHUNKS),
        in_specs=[
            pl.BlockSpec(memory_space=pltpu.SMEM),
            cache_spec,
            cache_spec,
            token_spec,
            token_spec,
        ],
        out_specs=[cache_spec, cache_spec],
        out_shape=[out_sds, out_sds],
        compiler_params=pltpu.CompilerParams(
            dimension_semantics=("arbitrary", "arbitrary"),
        ),
    )(lengths, cached_key, cached_value, key_token, value_token)
    return (new_key, new_value)


# speedup vs baseline: 47.9431x; 1.0821x over previous
import jax
import jax.numpy as jnp
from jax.experimental import pallas as pl
from jax.experimental.pallas import tpu as pltpu

B, S, H, D = 8, 2048, 8, 128
S_CHUNKS = 4
CS = S // S_CHUNKS


def _kv_append_kernel(len_ref, ck, cv, kt, vt, ok, ov):
    b = pl.program_id(0)
    c = pl.program_id(1)
    ok[...] = ck[...]
    ov[...] = cv[...]
    l = len_ref[b]
    base = c * CS

    @pl.when((l >= base) & (l < base + CS))
    def _():
        r = l - base
        ok[0, pl.ds(r, 1)] = kt[pl.ds(b, 1), 0]
        ov[0, pl.ds(r, 1)] = vt[pl.ds(b, 1), 0]


def kernel(cached_key, cached_value, key_token, value_token, lengths):
    out_sds = jax.ShapeDtypeStruct((B, S, H, D), jnp.float32)
    cache_spec = pl.BlockSpec((1, CS, H, D), lambda b, c: (b, c, 0, 0))
    token_spec = pl.BlockSpec((B, 1, H, D), lambda b, c: (0, 0, 0, 0))
    new_key, new_value = pl.pallas_call(
        _kv_append_kernel,
        grid=(B, S_CHUNKS),
        in_specs=[
            pl.BlockSpec(memory_space=pltpu.SMEM),
            cache_spec,
            cache_spec,
            token_spec,
            token_spec,
        ],
        out_specs=[cache_spec, cache_spec],
        out_shape=[out_sds, out_sds],
        compiler_params=pltpu.CompilerParams(
            dimension_semantics=("arbitrary", "arbitrary"),
        ),
    )(lengths, cached_key, cached_value, key_token, value_token)
    return (new_key, new_value)


# speedup vs baseline: 48.7616x; 1.0171x over previous
import jax
import jax.numpy as jnp
from jax.experimental import pallas as pl
from jax.experimental.pallas import tpu as pltpu

B, S, H, D = 8, 2048, 8, 128
S_CHUNKS = 2
CS = S // S_CHUNKS


def _kv_append_kernel(len_ref, ck, cv, kt, vt, ok, ov):
    b = pl.program_id(0)
    c = pl.program_id(1)
    ok[...] = ck[...]
    ov[...] = cv[...]
    l = len_ref[b]
    base = c * CS

    @pl.when((l >= base) & (l < base + CS))
    def _():
        r = l - base
        ok[0, pl.ds(r, 1)] = kt[pl.ds(b, 1), 0]
        ov[0, pl.ds(r, 1)] = vt[pl.ds(b, 1), 0]


def kernel(cached_key, cached_value, key_token, value_token, lengths):
    out_sds = jax.ShapeDtypeStruct((B, S, H, D), jnp.float32)
    cache_spec = pl.BlockSpec((1, CS, H, D), lambda b, c: (b, c, 0, 0))
    token_spec = pl.BlockSpec((B, 1, H, D), lambda b, c: (0, 0, 0, 0))
    new_key, new_value = pl.pallas_call(
        _kv_append_kernel,
        grid=(B, S_CHUNKS),
        in_specs=[
            pl.BlockSpec(memory_space=pltpu.SMEM),
            cache_spec,
            cache_spec,
            token_spec,
            token_spec,
        ],
        out_specs=[cache_spec, cache_spec],
        out_shape=[out_sds, out_sds],
        compiler_params=pltpu.CompilerParams(
            dimension_semantics=("arbitrary", "arbitrary"),
        ),
    )(lengths, cached_key, cached_value, key_token, value_token)
    return (new_key, new_value)


# speedup vs baseline: 48.8195x; 1.0012x over previous
import jax
import jax.numpy as jnp
from jax.experimental import pallas as pl
from jax.experimental.pallas import tpu as pltpu

B, S, H, D = 8, 2048, 8, 128
S_CHUNKS = 2
CS = S // S_CHUNKS


def _kv_append_kernel(len_ref, ck, cv, kt, vt, ok, ov):
    b = pl.program_id(0)
    c = pl.program_id(1)
    ok[...] = ck[...]
    ov[...] = cv[...]
    l = len_ref[b]
    base = c * CS

    @pl.when((l >= base) & (l < base + CS))
    def _():
        r = l - base
        ok[0, pl.ds(r, 1)] = kt[pl.ds(b, 1), 0]
        ov[0, pl.ds(r, 1)] = vt[pl.ds(b, 1), 0]


def kernel(cached_key, cached_value, key_token, value_token, lengths):
    out_sds = jax.ShapeDtypeStruct((B, S, H, D), jnp.float32)
    cache_spec = pl.BlockSpec((1, CS, H, D), lambda b, c: (b, c, 0, 0))
    token_spec = pl.BlockSpec((B, 1, H, D), lambda b, c: (0, 0, 0, 0))
    new_key, new_value = pl.pallas_call(
        _kv_append_kernel,
        grid=(B, S_CHUNKS),
        in_specs=[
            pl.BlockSpec(memory_space=pltpu.SMEM),
            cache_spec,
            cache_spec,
            token_spec,
            token_spec,
        ],
        out_specs=[cache_spec, cache_spec],
        out_shape=[out_sds, out_sds],
        compiler_params=pltpu.CompilerParams(
            dimension_semantics=("parallel", "parallel"),
            vmem_limit_bytes=100 * 1024 * 1024,
        ),
    )(lengths, cached_key, cached_value, key_token, value_token)
    return (new_key, new_value)
